# baseline (speedup 1.0000x reference)
import jax
import jax.numpy as jnp
from jax import lax
from jax.experimental import pallas as pl
from jax.experimental.pallas import tpu as pltpu
from jax.experimental.pallas import tpu_sc as plsc

N = 10000
E = 320000
D = 128
HID = 128
OUT = 64
L = 3
G = 16

NW = 32
N_PER_W = 320
N_PAD = NW * N_PER_W
N_CH = 5
E_CH = 79
E_PER_W = E_CH * 128
E_PAD = NW * E_PER_W
TPW = N_PAD // 16
RB = 512
N_BLK = N_PAD // RB


def _sc_counts_body(dst_hbm, z_hbm, o_hbm, cnt_hbm, didx_v, buf_v, cnt_sh):
    c = lax.axis_index("c")
    s = lax.axis_index("s")
    wid = s * 2 + c
    pltpu.sync_copy(dst_hbm.at[wid], didx_v)
    pltpu.sync_copy(z_hbm, cnt_sh.at[pl.ds(s * TPW, TPW)])
    pltpu.sync_copy(o_hbm, buf_v)
    plsc.subcore_barrier()

    def _cnt(j, _):
        pltpu.sync_copy(buf_v, cnt_sh.at[didx_v.at[j]], add=True)
        return 0
    lax.fori_loop(0, E_CH, _cnt, 0)
    plsc.subcore_barrier()
    pltpu.sync_copy(cnt_sh.at[pl.ds(s * TPW, TPW)],
                    cnt_hbm.at[c, pl.ds(s * TPW, TPW)])


def _sc_encode_body(xi_hbm, emb_hbm, h0_hbm, idx_v, iota_v, tmp_v,
                    emb_sh, acc_sh, sem):
    c = lax.axis_index("c")
    s = lax.axis_index("s")
    wid = s * 2 + c
    pltpu.sync_copy(xi_hbm.at[wid], idx_v)
    pltpu.sync_copy(emb_hbm.at[pl.ds(s * 64, 64)], tmp_v)
    pltpu.sync_copy(tmp_v, emb_sh.at[pl.ds(s * 64, 64)])
    plsc.subcore_barrier()

    i16 = lax.broadcasted_iota(jnp.int32, (16,), 0)
    win = s * 64
    for t in range(4):
        iota_v[0, pl.ds(t * 16, 16)] = i16 + (win + t * 16)
    for ch in range(N_CH):
        pltpu.async_copy(
            emb_sh.at[idx_v.at[0, pl.ds(ch * 64, 64)]], tmp_v, sem).wait()
        pltpu.sync_copy(tmp_v, acc_sh.at[pl.ds(win, 64)])

        def _k(k, _):
            pltpu.async_copy(
                emb_sh.at[idx_v.at[k, pl.ds(ch * 64, 64)]], tmp_v, sem).wait()
            pltpu.sync_copy(tmp_v, acc_sh.at[iota_v.at[0]], add=True)
            return 0
        lax.fori_loop(1, 9, _k, 0)
        pltpu.sync_copy(acc_sh.at[pl.ds(win, 64)], tmp_v)
        pltpu.sync_copy(tmp_v, h0_hbm.at[pl.ds(wid * N_PER_W + ch * 64, 64)])


def _sc_segsum_body(h_hbm, src_hbm, dst_hbm, z_hbm, part_hbm,
                    sidx_v, didx_v, rows0, acc_sh, sem0):
    c = lax.axis_index("c")
    s = lax.axis_index("s")
    wid = s * 2 + c
    pltpu.sync_copy(src_hbm.at[wid], sidx_v)
    pltpu.sync_copy(dst_hbm.at[wid], didx_v)

    pltpu.sync_copy(z_hbm, acc_sh.at[pl.ds(s * TPW, TPW)])
    plsc.subcore_barrier()

    def _step(j, _):
        pltpu.async_copy(
            h_hbm.at[sidx_v.at[pl.ds(j * 128, 128)]], rows0, sem0).wait()
        pltpu.sync_copy(rows0, acc_sh.at[didx_v.at[j]], add=True)
        return 0
    lax.fori_loop(0, E_CH, _step, 0)
    plsc.subcore_barrier()

    pltpu.sync_copy(acc_sh.at[pl.ds(s * TPW, TPW)],
                    part_hbm.at[c, pl.ds(s * TPW, TPW)])


def _tc_b_body(h0_ref, w1_ref, b1_ref, cnt_ref, h_ref, inv_ref):
    h_ref[...] = (jnp.dot(h0_ref[...], w1_ref[...],
                          preferred_element_type=jnp.float32) + b1_ref[...])
    csum = cnt_ref[0, :, 0:1] + cnt_ref[1, :, 0:1]
    inv_ref[...] = 1.0 / jnp.maximum(csum, 1.0)


def _tc_layer_body(part_ref, inv_ref, h_ref, wl_ref, wr_ref, b_ref, g_ref,
                   be_ref, out_ref):
    mean = (part_ref[0] + part_ref[1]) * inv_ref[...]
    z = (jnp.dot(mean, wl_ref[...], preferred_element_type=jnp.float32)
         + jnp.dot(h_ref[...], wr_ref[...], preferred_element_type=jnp.float32)
         + b_ref[...])
    z = jnp.maximum(z, 0.0)
    mu = jnp.mean(z, axis=1, keepdims=True)
    var = jnp.mean((z - mu) ** 2, axis=1, keepdims=True)
    out_ref[...] = (z - mu) * lax.rsqrt(var + 1e-5) * g_ref[...] + be_ref[...]


def _tc_pool_body(h_ref, w2_ref, b2_ref, bid_ref, out_ref):
    i = pl.program_id(0)
    z = (jnp.dot(h_ref[...], w2_ref[...],
                 preferred_element_type=jnp.float32) + b2_ref[...])
    rows = i * RB + lax.broadcasted_iota(jnp.int32, (RB, 1), 0)
    z = jnp.where(rows < N, z, -jnp.inf)
    bid = bid_ref[...]

    @pl.when(i == 0)
    def _init():
        out_ref[...] = jnp.full((G, OUT), -jnp.inf, jnp.float32)

    maxes = jnp.stack(
        [jnp.max(jnp.where(bid == g, z, -jnp.inf), axis=0) for g in range(G)])
    out_ref[...] = jnp.maximum(out_ref[...], maxes)

    @pl.when(i == pl.num_programs(0) - 1)
    def _fin():
        p = out_ref[...]
        out_ref[...] = 1.0 / (1.0 + jnp.exp(-p))


def kernel(x, edge_index, edge_attr, batch, size, params):
    del edge_attr
    x = x.astype(jnp.int32)
    xi = x + (jnp.arange(9, dtype=jnp.int32) * 100)[None, :]
    xpad = (jnp.arange(N_PAD - N, dtype=jnp.int32)[:, None] % 100
            + (jnp.arange(9, dtype=jnp.int32) * 100)[None, :])
    xi = jnp.concatenate([xi, xpad], axis=0)
    xi_w = xi.T.reshape(9, NW, N_PER_W).transpose(1, 0, 2)
    emb_pad = jnp.pad(params['node_emb'].reshape(9 * 100, D),
                      ((0, 124), (0, 0)))

    src = edge_index[0].astype(jnp.int32)
    dst = edge_index[1].astype(jnp.int32)
    ar = jnp.arange(E_PAD - E, dtype=jnp.int32)
    src_p = jnp.concatenate([src, ar % N])
    dst_p = jnp.concatenate([dst, N + ar % (N_PAD - N)])
    src2 = src_p.reshape(NW, E_PER_W)
    dst3 = dst_p.reshape(NW, E_CH, 128)
    zrows = jnp.zeros((TPW, D), jnp.float32)
    orows = jnp.ones((128, D), jnp.float32)

    mesh = plsc.VectorSubcoreMesh(core_axis_name="c", subcore_axis_name="s")

    counts = pl.kernel(
        _sc_counts_body, mesh=mesh,
        out_type=jax.ShapeDtypeStruct((2, N_PAD, D), jnp.float32),
        scratch_types=[pltpu.VMEM((E_CH, 128), jnp.int32),
                       pltpu.VMEM((128, D), jnp.float32),
                       pltpu.VMEM_SHARED((N_PAD, D), jnp.float32)])
    cnt_part = counts(dst3, zrows, orows)

    enc = pl.kernel(
        _sc_encode_body, mesh=mesh,
        out_type=jax.ShapeDtypeStruct((N_PAD, D), jnp.float32),
        scratch_types=[pltpu.VMEM((9, N_PER_W), jnp.int32),
                       pltpu.VMEM((1, 64), jnp.int32),
                       pltpu.VMEM((64, D), jnp.float32),
                       pltpu.VMEM_SHARED((1024, D), jnp.float32),
                       pltpu.VMEM_SHARED((16 * 64, D), jnp.float32),
                       pltpu.SemaphoreType.DMA])
    h0 = enc(xi_w, emb_pad)

    segsum = pl.kernel(
        _sc_segsum_body, mesh=mesh,
        out_type=jax.ShapeDtypeStruct((2, N_PAD, D), jnp.float32),
        scratch_types=[pltpu.VMEM((E_PER_W,), jnp.int32),
                       pltpu.VMEM((E_CH, 128), jnp.int32),
                       pltpu.VMEM((128, D), jnp.float32),
                       pltpu.VMEM_SHARED((N_PAD, D), jnp.float32),
                       pltpu.SemaphoreType.DMA])

    b1 = params['b1'].reshape(1, HID)
    h, inv = pl.pallas_call(
        _tc_b_body,
        grid=(N_BLK,),
        in_specs=[pl.BlockSpec((RB, D), lambda i: (i, 0)),
                  pl.BlockSpec((D, HID), lambda i: (0, 0)),
                  pl.BlockSpec((1, HID), lambda i: (0, 0)),
                  pl.BlockSpec((2, RB, D), lambda i: (0, i, 0))],
        out_specs=[pl.BlockSpec((RB, HID), lambda i: (i, 0)),
                   pl.BlockSpec((RB, 1), lambda i: (i, 0))],
        out_shape=[jax.ShapeDtypeStruct((N_PAD, HID), jnp.float32),
                   jax.ShapeDtypeStruct((N_PAD, 1), jnp.float32)],
    )(h0, params['W1'], b1, cnt_part)

    for li in range(L):
        part = segsum(h, src2, dst3, zrows)
        h = pl.pallas_call(
            _tc_layer_body,
            grid=(N_BLK,),
            in_specs=[pl.BlockSpec((2, RB, HID), lambda i: (0, i, 0)),
                      pl.BlockSpec((RB, 1), lambda i: (i, 0)),
                      pl.BlockSpec((RB, HID), lambda i: (i, 0)),
                      pl.BlockSpec((HID, HID), lambda i: (0, 0)),
                      pl.BlockSpec((HID, HID), lambda i: (0, 0)),
                      pl.BlockSpec((1, HID), lambda i: (0, 0)),
                      pl.BlockSpec((1, HID), lambda i: (0, 0)),
                      pl.BlockSpec((1, HID), lambda i: (0, 0))],
            out_specs=pl.BlockSpec((RB, HID), lambda i: (i, 0)),
            out_shape=jax.ShapeDtypeStruct((N_PAD, HID), jnp.float32),
        )(part, inv, h, params['Wl'][li], params['Wr'][li],
          params['bs'][li].reshape(1, HID),
          params['gamma'][li].reshape(1, HID),
          params['beta'][li].reshape(1, HID))

    seg = batch.astype(jnp.int32) + (jnp.asarray(size, jnp.int32) - G)
    seg = jnp.pad(seg, (0, N_PAD - N), constant_values=G)
    b2 = params['b2'].reshape(1, OUT)
    pooled = pl.pallas_call(
        _tc_pool_body,
        grid=(N_BLK,),
        in_specs=[pl.BlockSpec((RB, HID), lambda i: (i, 0)),
                  pl.BlockSpec((HID, OUT), lambda i: (0, 0)),
                  pl.BlockSpec((1, OUT), lambda i: (0, 0)),
                  pl.BlockSpec((RB, 1), lambda i: (i, 0))],
        out_specs=pl.BlockSpec((G, OUT), lambda i: (0, 0)),
        out_shape=jax.ShapeDtypeStruct((G, OUT), jnp.float32),
    )(h, params['W2'], b2, seg.reshape(N_PAD, 1))
    return pooled


# speedup vs baseline: 1.0128x; 1.0128x over previous
import jax
import jax.numpy as jnp
from jax import lax
from jax.experimental import pallas as pl
from jax.experimental.pallas import tpu as pltpu
from jax.experimental.pallas import tpu_sc as plsc

N = 10000
E = 320000
D = 128
HID = 128
OUT = 64
L = 3
G = 16

NW = 32
N_PER_W = 320
N_PAD = NW * N_PER_W
N_CH = 5
E_CH = 79
E_PER_W = E_CH * 128
E_PAD = NW * E_PER_W
TPW = N_PAD // 16
RB = 512
N_BLK = N_PAD // RB


def _sc_counts_body(dst_hbm, z_hbm, o_hbm, cnt_hbm, didx_v, buf_v, cnt_sh):
    c = lax.axis_index("c")
    s = lax.axis_index("s")
    wid = s * 2 + c
    pltpu.sync_copy(dst_hbm.at[wid], didx_v)
    pltpu.sync_copy(z_hbm, cnt_sh.at[pl.ds(s * TPW, TPW)])
    pltpu.sync_copy(o_hbm, buf_v)
    plsc.subcore_barrier()

    def _cnt(j, _):
        pltpu.sync_copy(buf_v, cnt_sh.at[didx_v.at[j]], add=True)
        return 0
    lax.fori_loop(0, E_CH, _cnt, 0)
    plsc.subcore_barrier()
    pltpu.sync_copy(cnt_sh.at[pl.ds(s * TPW, TPW)],
                    cnt_hbm.at[c, pl.ds(s * TPW, TPW)])


def _sc_encode_body(xi_hbm, emb_hbm, h0_hbm, idx_v, iota_v, tmp_v,
                    emb_sh, acc_sh, sem):
    c = lax.axis_index("c")
    s = lax.axis_index("s")
    wid = s * 2 + c
    pltpu.sync_copy(xi_hbm.at[wid], idx_v)
    pltpu.sync_copy(emb_hbm.at[pl.ds(s * 64, 64)], tmp_v)
    pltpu.sync_copy(tmp_v, emb_sh.at[pl.ds(s * 64, 64)])
    plsc.subcore_barrier()

    i16 = lax.broadcasted_iota(jnp.int32, (16,), 0)
    win = s * 64
    for t in range(4):
        iota_v[0, pl.ds(t * 16, 16)] = i16 + (win + t * 16)
    for ch in range(N_CH):
        pltpu.async_copy(
            emb_sh.at[idx_v.at[0, pl.ds(ch * 64, 64)]], tmp_v, sem).wait()
        pltpu.sync_copy(tmp_v, acc_sh.at[pl.ds(win, 64)])

        def _k(k, _):
            pltpu.async_copy(
                emb_sh.at[idx_v.at[k, pl.ds(ch * 64, 64)]], tmp_v, sem).wait()
            pltpu.sync_copy(tmp_v, acc_sh.at[iota_v.at[0]], add=True)
            return 0
        lax.fori_loop(1, 9, _k, 0)
        pltpu.sync_copy(acc_sh.at[pl.ds(win, 64)], tmp_v)
        pltpu.sync_copy(tmp_v, h0_hbm.at[pl.ds(wid * N_PER_W + ch * 64, 64)])


def _sc_segsum_body(h_hbm, src_hbm, dst_hbm, z_hbm, part_hbm,
                    sidx_v, didx_v, rows0, acc_sh, sem0):
    c = lax.axis_index("c")
    s = lax.axis_index("s")
    wid = s * 2 + c
    pltpu.sync_copy(src_hbm.at[wid], sidx_v)
    pltpu.sync_copy(dst_hbm.at[wid], didx_v)

    pltpu.sync_copy(z_hbm, acc_sh.at[pl.ds(s * TPW, TPW)])
    plsc.subcore_barrier()

    def _step(j, _):
        pltpu.async_copy(
            h_hbm.at[sidx_v.at[pl.ds(j * 128, 128)]], rows0, sem0).wait()
        pltpu.sync_copy(rows0, acc_sh.at[didx_v.at[j]], add=True)
        return 0
    lax.fori_loop(0, E_CH, _step, 0)
    plsc.subcore_barrier()

    pltpu.sync_copy(acc_sh.at[pl.ds(s * TPW, TPW)],
                    part_hbm.at[c, pl.ds(s * TPW, TPW)])


def _tc_b_body(h0_ref, w1_ref, b1_ref, cnt_ref, h_ref, inv_ref):
    h_ref[...] = (jnp.dot(h0_ref[...], w1_ref[...],
                          preferred_element_type=jnp.float32) + b1_ref[...])
    csum = cnt_ref[0, :, 0:1] + cnt_ref[1, :, 0:1]
    inv_ref[...] = 1.0 / jnp.maximum(csum, 1.0)


def _tc_layer_body(part_ref, inv_ref, h_ref, wl_ref, wr_ref, b_ref, g_ref,
                   be_ref, out_ref):
    mean = (part_ref[0] + part_ref[1]) * inv_ref[...]
    z = (jnp.dot(mean, wl_ref[...], preferred_element_type=jnp.float32)
         + jnp.dot(h_ref[...], wr_ref[...], preferred_element_type=jnp.float32)
         + b_ref[...])
    z = jnp.maximum(z, 0.0)
    mu = jnp.mean(z, axis=1, keepdims=True)
    var = jnp.mean((z - mu) ** 2, axis=1, keepdims=True)
    out_ref[...] = (z - mu) * lax.rsqrt(var + 1e-5) * g_ref[...] + be_ref[...]


def _tc_layer3_pool_body(part_ref, inv_ref, h_ref, wl_ref, wr_ref, b_ref,
                         g_ref, be_ref, w2_ref, b2_ref, bid_ref, out_ref):
    i = pl.program_id(0)
    mean = (part_ref[0] + part_ref[1]) * inv_ref[...]
    z = (jnp.dot(mean, wl_ref[...], preferred_element_type=jnp.float32)
         + jnp.dot(h_ref[...], wr_ref[...], preferred_element_type=jnp.float32)
         + b_ref[...])
    z = jnp.maximum(z, 0.0)
    mu = jnp.mean(z, axis=1, keepdims=True)
    var = jnp.mean((z - mu) ** 2, axis=1, keepdims=True)
    hn = (z - mu) * lax.rsqrt(var + 1e-5) * g_ref[...] + be_ref[...]

    zz = (jnp.dot(hn, w2_ref[...], preferred_element_type=jnp.float32)
          + b2_ref[...])
    rows = i * RB + lax.broadcasted_iota(jnp.int32, (RB, 1), 0)
    zz = jnp.where(rows < N, zz, -jnp.inf)
    bid = bid_ref[...]

    @pl.when(i == 0)
    def _init():
        out_ref[...] = jnp.full((G, OUT), -jnp.inf, jnp.float32)

    maxes = jnp.stack(
        [jnp.max(jnp.where(bid == g, zz, -jnp.inf), axis=0) for g in range(G)])
    out_ref[...] = jnp.maximum(out_ref[...], maxes)

    @pl.when(i == pl.num_programs(0) - 1)
    def _fin():
        p = out_ref[...]
        out_ref[...] = 1.0 / (1.0 + jnp.exp(-p))


def kernel(x, edge_index, edge_attr, batch, size, params):
    del edge_attr
    x = x.astype(jnp.int32)
    xi = x + (jnp.arange(9, dtype=jnp.int32) * 100)[None, :]
    xpad = (jnp.arange(N_PAD - N, dtype=jnp.int32)[:, None] % 100
            + (jnp.arange(9, dtype=jnp.int32) * 100)[None, :])
    xi = jnp.concatenate([xi, xpad], axis=0)
    xi_w = xi.T.reshape(9, NW, N_PER_W).transpose(1, 0, 2)
    emb_pad = jnp.pad(params['node_emb'].reshape(9 * 100, D),
                      ((0, 124), (0, 0)))

    src = edge_index[0].astype(jnp.int32)
    dst = edge_index[1].astype(jnp.int32)
    ar = jnp.arange(E_PAD - E, dtype=jnp.int32)
    src_p = jnp.concatenate([src, ar % N])
    dst_p = jnp.concatenate([dst, N + ar % (N_PAD - N)])
    src2 = src_p.reshape(NW, E_PER_W)
    dst3 = dst_p.reshape(NW, E_CH, 128)
    zrows = jnp.zeros((TPW, D), jnp.float32)
    orows = jnp.ones((128, D), jnp.float32)

    mesh = plsc.VectorSubcoreMesh(core_axis_name="c", subcore_axis_name="s")

    counts = pl.kernel(
        _sc_counts_body, mesh=mesh,
        out_type=jax.ShapeDtypeStruct((2, N_PAD, D), jnp.float32),
        scratch_types=[pltpu.VMEM((E_CH, 128), jnp.int32),
                       pltpu.VMEM((128, D), jnp.float32),
                       pltpu.VMEM_SHARED((N_PAD, D), jnp.float32)])
    cnt_part = counts(dst3, zrows, orows)

    enc = pl.kernel(
        _sc_encode_body, mesh=mesh,
        out_type=jax.ShapeDtypeStruct((N_PAD, D), jnp.float32),
        scratch_types=[pltpu.VMEM((9, N_PER_W), jnp.int32),
                       pltpu.VMEM((1, 64), jnp.int32),
                       pltpu.VMEM((64, D), jnp.float32),
                       pltpu.VMEM_SHARED((1024, D), jnp.float32),
                       pltpu.VMEM_SHARED((16 * 64, D), jnp.float32),
                       pltpu.SemaphoreType.DMA])
    h0 = enc(xi_w, emb_pad)

    segsum = pl.kernel(
        _sc_segsum_body, mesh=mesh,
        out_type=jax.ShapeDtypeStruct((2, N_PAD, D), jnp.float32),
        scratch_types=[pltpu.VMEM((E_PER_W,), jnp.int32),
                       pltpu.VMEM((E_CH, 128), jnp.int32),
                       pltpu.VMEM((128, D), jnp.float32),
                       pltpu.VMEM_SHARED((N_PAD, D), jnp.float32),
                       pltpu.SemaphoreType.DMA])

    b1 = params['b1'].reshape(1, HID)
    h, inv = pl.pallas_call(
        _tc_b_body,
        grid=(N_BLK,),
        in_specs=[pl.BlockSpec((RB, D), lambda i: (i, 0)),
                  pl.BlockSpec((D, HID), lambda i: (0, 0)),
                  pl.BlockSpec((1, HID), lambda i: (0, 0)),
                  pl.BlockSpec((2, RB, D), lambda i: (0, i, 0))],
        out_specs=[pl.BlockSpec((RB, HID), lambda i: (i, 0)),
                   pl.BlockSpec((RB, 1), lambda i: (i, 0))],
        out_shape=[jax.ShapeDtypeStruct((N_PAD, HID), jnp.float32),
                   jax.ShapeDtypeStruct((N_PAD, 1), jnp.float32)],
    )(h0, params['W1'], b1, cnt_part)

    for li in range(L - 1):
        part = segsum(h, src2, dst3, zrows)
        h = pl.pallas_call(
            _tc_layer_body,
            grid=(N_BLK,),
            in_specs=[pl.BlockSpec((2, RB, HID), lambda i: (0, i, 0)),
                      pl.BlockSpec((RB, 1), lambda i: (i, 0)),
                      pl.BlockSpec((RB, HID), lambda i: (i, 0)),
                      pl.BlockSpec((HID, HID), lambda i: (0, 0)),
                      pl.BlockSpec((HID, HID), lambda i: (0, 0)),
                      pl.BlockSpec((1, HID), lambda i: (0, 0)),
                      pl.BlockSpec((1, HID), lambda i: (0, 0)),
                      pl.BlockSpec((1, HID), lambda i: (0, 0))],
            out_specs=pl.BlockSpec((RB, HID), lambda i: (i, 0)),
            out_shape=jax.ShapeDtypeStruct((N_PAD, HID), jnp.float32),
        )(part, inv, h, params['Wl'][li], params['Wr'][li],
          params['bs'][li].reshape(1, HID),
          params['gamma'][li].reshape(1, HID),
          params['beta'][li].reshape(1, HID))

    part = segsum(h, src2, dst3, zrows)
    seg = batch.astype(jnp.int32) + (jnp.asarray(size, jnp.int32) - G)
    seg = jnp.pad(seg, (0, N_PAD - N), constant_values=G)
    b2 = params['b2'].reshape(1, OUT)
    li = L - 1
    pooled = pl.pallas_call(
        _tc_layer3_pool_body,
        grid=(N_BLK,),
        in_specs=[pl.BlockSpec((2, RB, HID), lambda i: (0, i, 0)),
                  pl.BlockSpec((RB, 1), lambda i: (i, 0)),
                  pl.BlockSpec((RB, HID), lambda i: (i, 0)),
                  pl.BlockSpec((HID, HID), lambda i: (0, 0)),
                  pl.BlockSpec((HID, HID), lambda i: (0, 0)),
                  pl.BlockSpec((1, HID), lambda i: (0, 0)),
                  pl.BlockSpec((1, HID), lambda i: (0, 0)),
                  pl.BlockSpec((1, HID), lambda i: (0, 0)),
                  pl.BlockSpec((HID, OUT), lambda i: (0, 0)),
                  pl.BlockSpec((1, OUT), lambda i: (0, 0)),
                  pl.BlockSpec((RB, 1), lambda i: (i, 0))],
        out_specs=pl.BlockSpec((G, OUT), lambda i: (0, 0)),
        out_shape=jax.ShapeDtypeStruct((G, OUT), jnp.float32),
    )(part, inv, h, params['Wl'][li], params['Wr'][li],
      params['bs'][li].reshape(1, HID),
      params['gamma'][li].reshape(1, HID),
      params['beta'][li].reshape(1, HID),
      params['W2'], b2, seg.reshape(N_PAD, 1))
    return pooled


# speedup vs baseline: 1.1178x; 1.1036x over previous
import jax
import jax.numpy as jnp
from jax import lax
from jax.experimental import pallas as pl
from jax.experimental.pallas import tpu as pltpu
from jax.experimental.pallas import tpu_sc as plsc

N = 10000
E = 320000
D = 128
HID = 128
OUT = 64
L = 3
G = 16

NW = 32
N_PER_W = 320
N_PAD = NW * N_PER_W
N_CH = 5
E_CH = 79
E_PER_W = E_CH * 128
E_PAD = NW * E_PER_W
TPW = N_PAD // 16
RB = 512
N_BLK = N_PAD // RB


def _sc_counts_body(dst_hbm, z_hbm, o_hbm, cnt_hbm, didx_v, buf_v, cnt_sh):
    c = lax.axis_index("c")
    s = lax.axis_index("s")
    wid = s * 2 + c
    pltpu.sync_copy(dst_hbm.at[wid], didx_v)
    pltpu.sync_copy(z_hbm, cnt_sh.at[pl.ds(s * TPW, TPW)])
    pltpu.sync_copy(o_hbm, buf_v)
    plsc.subcore_barrier()

    def _cnt(j, _):
        pltpu.sync_copy(buf_v, cnt_sh.at[didx_v.at[j]], add=True)
        return 0
    lax.fori_loop(0, E_CH, _cnt, 0)
    plsc.subcore_barrier()
    pltpu.sync_copy(cnt_sh.at[pl.ds(s * TPW, TPW)],
                    cnt_hbm.at[c, pl.ds(s * TPW, TPW)])


def _sc_encode_body(xi_hbm, emb_hbm, h0_hbm, idx_v, iota_v, tmp_v,
                    emb_sh, acc_sh, sem):
    c = lax.axis_index("c")
    s = lax.axis_index("s")
    wid = s * 2 + c
    pltpu.sync_copy(xi_hbm.at[wid], idx_v)
    pltpu.sync_copy(emb_hbm.at[pl.ds(s * 64, 64)], tmp_v)
    pltpu.sync_copy(tmp_v, emb_sh.at[pl.ds(s * 64, 64)])
    plsc.subcore_barrier()

    i16 = lax.broadcasted_iota(jnp.int32, (16,), 0)
    win = s * 64
    for t in range(4):
        iota_v[0, pl.ds(t * 16, 16)] = i16 + (win + t * 16)
    for ch in range(N_CH):
        pltpu.async_copy(
            emb_sh.at[idx_v.at[0, pl.ds(ch * 64, 64)]], tmp_v, sem).wait()
        pltpu.sync_copy(tmp_v, acc_sh.at[pl.ds(win, 64)])

        def _k(k, _):
            pltpu.async_copy(
                emb_sh.at[idx_v.at[k, pl.ds(ch * 64, 64)]], tmp_v, sem).wait()
            pltpu.sync_copy(tmp_v, acc_sh.at[iota_v.at[0]], add=True)
            return 0
        lax.fori_loop(1, 9, _k, 0)
        pltpu.sync_copy(acc_sh.at[pl.ds(win, 64)], tmp_v)
        pltpu.sync_copy(tmp_v, h0_hbm.at[pl.ds(wid * N_PER_W + ch * 64, 64)])


def _sc_segsum_body(h_hbm, src_hbm, dst_hbm, z_hbm, part_hbm,
                    sidx_v, didx_v, rows0, rows1, acc_sh, sem0, sem1):
    c = lax.axis_index("c")
    s = lax.axis_index("s")
    wid = s * 2 + c
    pltpu.sync_copy(src_hbm.at[wid], sidx_v)
    pltpu.sync_copy(dst_hbm.at[wid, pl.ds(0, 40)], didx_v)

    pltpu.sync_copy(z_hbm, acc_sh.at[pl.ds(s * TPW, TPW)])
    plsc.subcore_barrier()

    def _pair(p, off):
        j = 2 * p
        ca = pltpu.async_copy(
            h_hbm.at[sidx_v.at[pl.ds((off + j) * 128, 128)]], rows0, sem0)
        cb = pltpu.async_copy(
            h_hbm.at[sidx_v.at[pl.ds((off + j + 1) * 128, 128)]], rows1, sem1)
        ca.wait()
        pltpu.sync_copy(rows0, acc_sh.at[didx_v.at[j]], add=True)
        cb.wait()
        pltpu.sync_copy(rows1, acc_sh.at[didx_v.at[j + 1]], add=True)

    def _stepa(p, _):
        _pair(p, 0)
        return 0
    lax.fori_loop(0, 20, _stepa, 0)
    pltpu.sync_copy(dst_hbm.at[wid, pl.ds(40, 39)],
                    didx_v.at[pl.ds(0, 39)])

    def _stepb(p, _):
        _pair(p, 40)
        return 0
    lax.fori_loop(0, 19, _stepb, 0)
    pltpu.async_copy(
        h_hbm.at[sidx_v.at[pl.ds(78 * 128, 128)]], rows0, sem0).wait()
    pltpu.sync_copy(rows0, acc_sh.at[didx_v.at[38]], add=True)
    plsc.subcore_barrier()

    pltpu.sync_copy(acc_sh.at[pl.ds(s * TPW, TPW)],
                    part_hbm.at[c, pl.ds(s * TPW, TPW)])


def _tc_b_body(h0_ref, w1_ref, b1_ref, cnt_ref, h_ref, inv_ref):
    h_ref[...] = (jnp.dot(h0_ref[...], w1_ref[...],
                          preferred_element_type=jnp.float32) + b1_ref[...])
    csum = cnt_ref[0, :, 0:1] + cnt_ref[1, :, 0:1]
    inv_ref[...] = 1.0 / jnp.maximum(csum, 1.0)


def _tc_layer_body(part_ref, inv_ref, h_ref, wl_ref, wr_ref, b_ref, g_ref,
                   be_ref, out_ref):
    mean = (part_ref[0] + part_ref[1]) * inv_ref[...]
    z = (jnp.dot(mean, wl_ref[...], preferred_element_type=jnp.float32)
         + jnp.dot(h_ref[...], wr_ref[...], preferred_element_type=jnp.float32)
         + b_ref[...])
    z = jnp.maximum(z, 0.0)
    mu = jnp.mean(z, axis=1, keepdims=True)
    var = jnp.mean((z - mu) ** 2, axis=1, keepdims=True)
    out_ref[...] = (z - mu) * lax.rsqrt(var + 1e-5) * g_ref[...] + be_ref[...]


def _tc_layer3_pool_body(part_ref, inv_ref, h_ref, wl_ref, wr_ref, b_ref,
                         g_ref, be_ref, w2_ref, b2_ref, bid_ref, out_ref):
    i = pl.program_id(0)
    mean = (part_ref[0] + part_ref[1]) * inv_ref[...]
    z = (jnp.dot(mean, wl_ref[...], preferred_element_type=jnp.float32)
         + jnp.dot(h_ref[...], wr_ref[...], preferred_element_type=jnp.float32)
         + b_ref[...])
    z = jnp.maximum(z, 0.0)
    mu = jnp.mean(z, axis=1, keepdims=True)
    var = jnp.mean((z - mu) ** 2, axis=1, keepdims=True)
    hn = (z - mu) * lax.rsqrt(var + 1e-5) * g_ref[...] + be_ref[...]

    zz = (jnp.dot(hn, w2_ref[...], preferred_element_type=jnp.float32)
          + b2_ref[...])
    rows = i * RB + lax.broadcasted_iota(jnp.int32, (RB, 1), 0)
    zz = jnp.where(rows < N, zz, -jnp.inf)
    bid = bid_ref[...]

    @pl.when(i == 0)
    def _init():
        out_ref[...] = jnp.full((G, OUT), -jnp.inf, jnp.float32)

    maxes = jnp.stack(
        [jnp.max(jnp.where(bid == g, zz, -jnp.inf), axis=0) for g in range(G)])
    out_ref[...] = jnp.maximum(out_ref[...], maxes)

    @pl.when(i == pl.num_programs(0) - 1)
    def _fin():
        p = out_ref[...]
        out_ref[...] = 1.0 / (1.0 + jnp.exp(-p))


def kernel(x, edge_index, edge_attr, batch, size, params):
    del edge_attr
    x = x.astype(jnp.int32)
    xi = x + (jnp.arange(9, dtype=jnp.int32) * 100)[None, :]
    xpad = (jnp.arange(N_PAD - N, dtype=jnp.int32)[:, None] % 100
            + (jnp.arange(9, dtype=jnp.int32) * 100)[None, :])
    xi = jnp.concatenate([xi, xpad], axis=0)
    xi_w = xi.T.reshape(9, NW, N_PER_W).transpose(1, 0, 2)
    emb_pad = jnp.pad(params['node_emb'].reshape(9 * 100, D),
                      ((0, 124), (0, 0)))

    src = edge_index[0].astype(jnp.int32)
    dst = edge_index[1].astype(jnp.int32)
    ar = jnp.arange(E_PAD - E, dtype=jnp.int32)
    src_p = jnp.concatenate([src, ar % N])
    dst_p = jnp.concatenate([dst, N + ar % (N_PAD - N)])
    src2 = src_p.reshape(NW, E_PER_W)
    dst3 = dst_p.reshape(NW, E_CH, 128)
    zrows = jnp.zeros((TPW, D), jnp.float32)
    orows = jnp.ones((128, D), jnp.float32)

    mesh = plsc.VectorSubcoreMesh(core_axis_name="c", subcore_axis_name="s")

    counts = pl.kernel(
        _sc_counts_body, mesh=mesh,
        out_type=jax.ShapeDtypeStruct((2, N_PAD, D), jnp.float32),
        scratch_types=[pltpu.VMEM((E_CH, 128), jnp.int32),
                       pltpu.VMEM((128, D), jnp.float32),
                       pltpu.VMEM_SHARED((N_PAD, D), jnp.float32)])
    cnt_part = counts(dst3, zrows, orows)

    enc = pl.kernel(
        _sc_encode_body, mesh=mesh,
        out_type=jax.ShapeDtypeStruct((N_PAD, D), jnp.float32),
        scratch_types=[pltpu.VMEM((9, N_PER_W), jnp.int32),
                       pltpu.VMEM((1, 64), jnp.int32),
                       pltpu.VMEM((64, D), jnp.float32),
                       pltpu.VMEM_SHARED((1024, D), jnp.float32),
                       pltpu.VMEM_SHARED((16 * 64, D), jnp.float32),
                       pltpu.SemaphoreType.DMA])
    h0 = enc(xi_w, emb_pad)

    segsum = pl.kernel(
        _sc_segsum_body, mesh=mesh,
        out_type=jax.ShapeDtypeStruct((2, N_PAD, D), jnp.float32),
        scratch_types=[pltpu.VMEM((E_PER_W,), jnp.int32),
                       pltpu.VMEM((40, 128), jnp.int32),
                       pltpu.VMEM((128, D), jnp.float32),
                       pltpu.VMEM((128, D), jnp.float32),
                       pltpu.VMEM_SHARED((N_PAD, D), jnp.float32),
                       pltpu.SemaphoreType.DMA,
                       pltpu.SemaphoreType.DMA])

    b1 = params['b1'].reshape(1, HID)
    h, inv = pl.pallas_call(
        _tc_b_body,
        grid=(N_BLK,),
        in_specs=[pl.BlockSpec((RB, D), lambda i: (i, 0)),
                  pl.BlockSpec((D, HID), lambda i: (0, 0)),
                  pl.BlockSpec((1, HID), lambda i: (0, 0)),
                  pl.BlockSpec((2, RB, D), lambda i: (0, i, 0))],
        out_specs=[pl.BlockSpec((RB, HID), lambda i: (i, 0)),
                   pl.BlockSpec((RB, 1), lambda i: (i, 0))],
        out_shape=[jax.ShapeDtypeStruct((N_PAD, HID), jnp.float32),
                   jax.ShapeDtypeStruct((N_PAD, 1), jnp.float32)],
    )(h0, params['W1'], b1, cnt_part)

    for li in range(L - 1):
        part = segsum(h, src2, dst3, zrows)
        h = pl.pallas_call(
            _tc_layer_body,
            grid=(N_BLK,),
            in_specs=[pl.BlockSpec((2, RB, HID), lambda i: (0, i, 0)),
                      pl.BlockSpec((RB, 1), lambda i: (i, 0)),
                      pl.BlockSpec((RB, HID), lambda i: (i, 0)),
                      pl.BlockSpec((HID, HID), lambda i: (0, 0)),
                      pl.BlockSpec((HID, HID), lambda i: (0, 0)),
                      pl.BlockSpec((1, HID), lambda i: (0, 0)),
                      pl.BlockSpec((1, HID), lambda i: (0, 0)),
                      pl.BlockSpec((1, HID), lambda i: (0, 0))],
            out_specs=pl.BlockSpec((RB, HID), lambda i: (i, 0)),
            out_shape=jax.ShapeDtypeStruct((N_PAD, HID), jnp.float32),
        )(part, inv, h, params['Wl'][li], params['Wr'][li],
          params['bs'][li].reshape(1, HID),
          params['gamma'][li].reshape(1, HID),
          params['beta'][li].reshape(1, HID))

    part = segsum(h, src2, dst3, zrows)
    seg = batch.astype(jnp.int32) + (jnp.asarray(size, jnp.int32) - G)
    seg = jnp.pad(seg, (0, N_PAD - N), constant_values=G)
    b2 = params['b2'].reshape(1, OUT)
    li = L - 1
    pooled = pl.pallas_call(
        _tc_layer3_pool_body,
        grid=(N_BLK,),
        in_specs=[pl.BlockSpec((2, RB, HID), lambda i: (0, i, 0)),
                  pl.BlockSpec((RB, 1), lambda i: (i, 0)),
                  pl.BlockSpec((RB, HID), lambda i: (i, 0)),
                  pl.BlockSpec((HID, HID), lambda i: (0, 0)),
                  pl.BlockSpec((HID, HID), lambda i: (0, 0)),
                  pl.BlockSpec((1, HID), lambda i: (0, 0)),
                  pl.BlockSpec((1, HID), lambda i: (0, 0)),
                  pl.BlockSpec((1, HID), lambda i: (0, 0)),
                  pl.BlockSpec((HID, OUT), lambda i: (0, 0)),
                  pl.BlockSpec((1, OUT), lambda i: (0, 0)),
                  pl.BlockSpec((RB, 1), lambda i: (i, 0))],
        out_specs=pl.BlockSpec((G, OUT), lambda i: (0, 0)),
        out_shape=jax.ShapeDtypeStruct((G, OUT), jnp.float32),
    )(part, inv, h, params['Wl'][li], params['Wr'][li],
      params['bs'][li].reshape(1, HID),
      params['gamma'][li].reshape(1, HID),
      params['beta'][li].reshape(1, HID),
      params['W2'], b2, seg.reshape(N_PAD, 1))
    return pooled


# speedup vs baseline: 1.1267x; 1.0079x over previous
import jax
import jax.numpy as jnp
from jax import lax
from jax.experimental import pallas as pl
from jax.experimental.pallas import tpu as pltpu
from jax.experimental.pallas import tpu_sc as plsc

N = 10000
E = 320000
D = 128
HID = 128
OUT = 64
L = 3
G = 16

NW = 32
N_PER_W = 320
N_PAD = NW * N_PER_W
N_CH = 5
E_CH = 79
E_PER_W = E_CH * 128
E_PAD = NW * E_PER_W
TPW = N_PAD // 16
RB = 512
N_BLK = N_PAD // RB


def _sc_counts_body(dst_hbm, z_hbm, o_hbm, cnt_hbm, didx_v, buf_v, cnt_sh):
    c = lax.axis_index("c")
    s = lax.axis_index("s")
    wid = s * 2 + c
    pltpu.sync_copy(dst_hbm.at[wid], didx_v)
    pltpu.sync_copy(z_hbm, cnt_sh.at[pl.ds(s * TPW, TPW)])
    pltpu.sync_copy(o_hbm, buf_v)
    plsc.subcore_barrier()

    def _cnt(j, _):
        pltpu.sync_copy(buf_v, cnt_sh.at[didx_v.at[j]], add=True)
        return 0
    lax.fori_loop(0, E_CH, _cnt, 0)
    plsc.subcore_barrier()
    pltpu.sync_copy(cnt_sh.at[pl.ds(s * TPW, TPW)],
                    cnt_hbm.at[c, pl.ds(s * TPW, TPW)])


def _sc_encode_body(xi_hbm, emb_hbm, h0_hbm, idx_v, iota_v, tmp_v, tmp2_v,
                    emb_sh, acc_sh, sem, sem1):
    c = lax.axis_index("c")
    s = lax.axis_index("s")
    wid = s * 2 + c
    pltpu.sync_copy(xi_hbm.at[wid], idx_v)
    pltpu.sync_copy(emb_hbm.at[pl.ds(s * 64, 64)], tmp_v)
    pltpu.sync_copy(tmp_v, emb_sh.at[pl.ds(s * 64, 64)])
    plsc.subcore_barrier()

    i16 = lax.broadcasted_iota(jnp.int32, (16,), 0)
    win = s * 64
    for t in range(4):
        iota_v[0, pl.ds(t * 16, 16)] = i16 + (win + t * 16)
    for ch in range(N_CH):
        pltpu.async_copy(
            emb_sh.at[idx_v.at[0, pl.ds(ch * 64, 64)]], tmp_v, sem).wait()
        pltpu.sync_copy(tmp_v, acc_sh.at[pl.ds(win, 64)])

        def _k(p, _):
            ka = 1 + 2 * p
            ca = pltpu.async_copy(
                emb_sh.at[idx_v.at[ka, pl.ds(ch * 64, 64)]], tmp_v, sem)
            cb = pltpu.async_copy(
                emb_sh.at[idx_v.at[ka + 1, pl.ds(ch * 64, 64)]], tmp2_v, sem1)
            ca.wait()
            pltpu.sync_copy(tmp_v, acc_sh.at[iota_v.at[0]], add=True)
            cb.wait()
            pltpu.sync_copy(tmp2_v, acc_sh.at[iota_v.at[0]], add=True)
            return 0
        lax.fori_loop(0, 4, _k, 0)
        pltpu.sync_copy(acc_sh.at[pl.ds(win, 64)],
                        h0_hbm.at[pl.ds(wid * N_PER_W + ch * 64, 64)])


def _sc_segsum_body(h_hbm, src_hbm, dst_hbm, z_hbm, part_hbm,
                    sidx_v, didx_v, rows0, rows1, acc_sh, sem0, sem1):
    c = lax.axis_index("c")
    s = lax.axis_index("s")
    wid = s * 2 + c
    pltpu.sync_copy(src_hbm.at[wid], sidx_v)
    pltpu.sync_copy(dst_hbm.at[wid, pl.ds(0, 40)], didx_v)

    pltpu.sync_copy(z_hbm, acc_sh.at[pl.ds(s * TPW, TPW)])
    plsc.subcore_barrier()

    def _pair(p, off):
        j = 2 * p
        ca = pltpu.async_copy(
            h_hbm.at[sidx_v.at[pl.ds((off + j) * 128, 128)]], rows0, sem0)
        cb = pltpu.async_copy(
            h_hbm.at[sidx_v.at[pl.ds((off + j + 1) * 128, 128)]], rows1, sem1)
        ca.wait()
        pltpu.sync_copy(rows0, acc_sh.at[didx_v.at[j]], add=True)
        cb.wait()
        pltpu.sync_copy(rows1, acc_sh.at[didx_v.at[j + 1]], add=True)

    def _stepa(p, _):
        _pair(p, 0)
        return 0
    lax.fori_loop(0, 20, _stepa, 0)
    pltpu.sync_copy(dst_hbm.at[wid, pl.ds(40, 39)],
                    didx_v.at[pl.ds(0, 39)])

    def _stepb(p, _):
        _pair(p, 40)
        return 0
    lax.fori_loop(0, 19, _stepb, 0)
    pltpu.async_copy(
        h_hbm.at[sidx_v.at[pl.ds(78 * 128, 128)]], rows0, sem0).wait()
    pltpu.sync_copy(rows0, acc_sh.at[didx_v.at[38]], add=True)
    plsc.subcore_barrier()

    pltpu.sync_copy(acc_sh.at[pl.ds(s * TPW, TPW)],
                    part_hbm.at[c, pl.ds(s * TPW, TPW)])


def _tc_b_body(h0_ref, w1_ref, b1_ref, cnt_ref, h_ref, inv_ref):
    h_ref[...] = (jnp.dot(h0_ref[...], w1_ref[...],
                          preferred_element_type=jnp.float32) + b1_ref[...])
    csum = cnt_ref[0, :, 0:1] + cnt_ref[1, :, 0:1]
    inv_ref[...] = 1.0 / jnp.maximum(csum, 1.0)


def _tc_layer_body(part_ref, inv_ref, h_ref, wl_ref, wr_ref, b_ref, g_ref,
                   be_ref, out_ref):
    mean = (part_ref[0] + part_ref[1]) * inv_ref[...]
    z = (jnp.dot(mean, wl_ref[...], preferred_element_type=jnp.float32)
         + jnp.dot(h_ref[...], wr_ref[...], preferred_element_type=jnp.float32)
         + b_ref[...])
    z = jnp.maximum(z, 0.0)
    mu = jnp.mean(z, axis=1, keepdims=True)
    var = jnp.mean((z - mu) ** 2, axis=1, keepdims=True)
    out_ref[...] = (z - mu) * lax.rsqrt(var + 1e-5) * g_ref[...] + be_ref[...]


def _tc_layer3_pool_body(part_ref, inv_ref, h_ref, wl_ref, wr_ref, b_ref,
                         g_ref, be_ref, w2_ref, b2_ref, bid_ref, out_ref):
    i = pl.program_id(0)
    mean = (part_ref[0] + part_ref[1]) * inv_ref[...]
    z = (jnp.dot(mean, wl_ref[...], preferred_element_type=jnp.float32)
         + jnp.dot(h_ref[...], wr_ref[...], preferred_element_type=jnp.float32)
         + b_ref[...])
    z = jnp.maximum(z, 0.0)
    mu = jnp.mean(z, axis=1, keepdims=True)
    var = jnp.mean((z - mu) ** 2, axis=1, keepdims=True)
    hn = (z - mu) * lax.rsqrt(var + 1e-5) * g_ref[...] + be_ref[...]

    zz = (jnp.dot(hn, w2_ref[...], preferred_element_type=jnp.float32)
          + b2_ref[...])
    rows = i * RB + lax.broadcasted_iota(jnp.int32, (RB, 1), 0)
    zz = jnp.where(rows < N, zz, -jnp.inf)
    bid = bid_ref[...]

    @pl.when(i == 0)
    def _init():
        out_ref[...] = jnp.full((G, OUT), -jnp.inf, jnp.float32)

    maxes = jnp.stack(
        [jnp.max(jnp.where(bid == g, zz, -jnp.inf), axis=0) for g in range(G)])
    out_ref[...] = jnp.maximum(out_ref[...], maxes)

    @pl.when(i == pl.num_programs(0) - 1)
    def _fin():
        p = out_ref[...]
        out_ref[...] = 1.0 / (1.0 + jnp.exp(-p))


def kernel(x, edge_index, edge_attr, batch, size, params):
    del edge_attr
    x = x.astype(jnp.int32)
    xi = x + (jnp.arange(9, dtype=jnp.int32) * 100)[None, :]
    xpad = (jnp.arange(N_PAD - N, dtype=jnp.int32)[:, None] % 100
            + (jnp.arange(9, dtype=jnp.int32) * 100)[None, :])
    xi = jnp.concatenate([xi, xpad], axis=0)
    xi_w = xi.T.reshape(9, NW, N_PER_W).transpose(1, 0, 2)
    emb_pad = jnp.pad(params['node_emb'].reshape(9 * 100, D),
                      ((0, 124), (0, 0)))

    src = edge_index[0].astype(jnp.int32)
    dst = edge_index[1].astype(jnp.int32)
    ar = jnp.arange(E_PAD - E, dtype=jnp.int32)
    src_p = jnp.concatenate([src, ar % N])
    dst_p = jnp.concatenate([dst, N + ar % (N_PAD - N)])
    src2 = src_p.reshape(NW, E_PER_W)
    dst3 = dst_p.reshape(NW, E_CH, 128)
    zrows = jnp.zeros((TPW, D), jnp.float32)
    orows = jnp.ones((128, D), jnp.float32)

    mesh = plsc.VectorSubcoreMesh(core_axis_name="c", subcore_axis_name="s")

    counts = pl.kernel(
        _sc_counts_body, mesh=mesh,
        out_type=jax.ShapeDtypeStruct((2, N_PAD, D), jnp.float32),
        scratch_types=[pltpu.VMEM((E_CH, 128), jnp.int32),
                       pltpu.VMEM((128, D), jnp.float32),
                       pltpu.VMEM_SHARED((N_PAD, D), jnp.float32)])
    cnt_part = counts(dst3, zrows, orows)

    enc = pl.kernel(
        _sc_encode_body, mesh=mesh,
        out_type=jax.ShapeDtypeStruct((N_PAD, D), jnp.float32),
        scratch_types=[pltpu.VMEM((9, N_PER_W), jnp.int32),
                       pltpu.VMEM((1, 64), jnp.int32),
                       pltpu.VMEM((64, D), jnp.float32),
                       pltpu.VMEM((64, D), jnp.float32),
                       pltpu.VMEM_SHARED((1024, D), jnp.float32),
                       pltpu.VMEM_SHARED((16 * 64, D), jnp.float32),
                       pltpu.SemaphoreType.DMA,
                       pltpu.SemaphoreType.DMA])
    h0 = enc(xi_w, emb_pad)

    segsum = pl.kernel(
        _sc_segsum_body, mesh=mesh,
        out_type=jax.ShapeDtypeStruct((2, N_PAD, D), jnp.float32),
        scratch_types=[pltpu.VMEM((E_PER_W,), jnp.int32),
                       pltpu.VMEM((40, 128), jnp.int32),
                       pltpu.VMEM((128, D), jnp.float32),
                       pltpu.VMEM((128, D), jnp.float32),
                       pltpu.VMEM_SHARED((N_PAD, D), jnp.float32),
                       pltpu.SemaphoreType.DMA,
                       pltpu.SemaphoreType.DMA])

    b1 = params['b1'].reshape(1, HID)
    h, inv = pl.pallas_call(
        _tc_b_body,
        grid=(N_BLK,),
        in_specs=[pl.BlockSpec((RB, D), lambda i: (i, 0)),
                  pl.BlockSpec((D, HID), lambda i: (0, 0)),
                  pl.BlockSpec((1, HID), lambda i: (0, 0)),
                  pl.BlockSpec((2, RB, D), lambda i: (0, i, 0))],
        out_specs=[pl.BlockSpec((RB, HID), lambda i: (i, 0)),
                   pl.BlockSpec((RB, 1), lambda i: (i, 0))],
        out_shape=[jax.ShapeDtypeStruct((N_PAD, HID), jnp.float32),
                   jax.ShapeDtypeStruct((N_PAD, 1), jnp.float32)],
    )(h0, params['W1'], b1, cnt_part)

    for li in range(L - 1):
        part = segsum(h, src2, dst3, zrows)
        h = pl.pallas_call(
            _tc_layer_body,
            grid=(N_BLK,),
            in_specs=[pl.BlockSpec((2, RB, HID), lambda i: (0, i, 0)),
                      pl.BlockSpec((RB, 1), lambda i: (i, 0)),
                      pl.BlockSpec((RB, HID), lambda i: (i, 0)),
                      pl.BlockSpec((HID, HID), lambda i: (0, 0)),
                      pl.BlockSpec((HID, HID), lambda i: (0, 0)),
                      pl.BlockSpec((1, HID), lambda i: (0, 0)),
                      pl.BlockSpec((1, HID), lambda i: (0, 0)),
                      pl.BlockSpec((1, HID), lambda i: (0, 0))],
            out_specs=pl.BlockSpec((RB, HID), lambda i: (i, 0)),
            out_shape=jax.ShapeDtypeStruct((N_PAD, HID), jnp.float32),
        )(part, inv, h, params['Wl'][li], params['Wr'][li],
          params['bs'][li].reshape(1, HID),
          params['gamma'][li].reshape(1, HID),
          params['beta'][li].reshape(1, HID))

    part = segsum(h, src2, dst3, zrows)
    seg = batch.astype(jnp.int32) + (jnp.asarray(size, jnp.int32) - G)
    seg = jnp.pad(seg, (0, N_PAD - N), constant_values=G)
    b2 = params['b2'].reshape(1, OUT)
    li = L - 1
    pooled = pl.pallas_call(
        _tc_layer3_pool_body,
        grid=(N_BLK,),
        in_specs=[pl.BlockSpec((2, RB, HID), lambda i: (0, i, 0)),
                  pl.BlockSpec((RB, 1), lambda i: (i, 0)),
                  pl.BlockSpec((RB, HID), lambda i: (i, 0)),
                  pl.BlockSpec((HID, HID), lambda i: (0, 0)),
                  pl.BlockSpec((HID, HID), lambda i: (0, 0)),
                  pl.BlockSpec((1, HID), lambda i: (0, 0)),
                  pl.BlockSpec((1, HID), lambda i: (0, 0)),
                  pl.BlockSpec((1, HID), lambda i: (0, 0)),
                  pl.BlockSpec((HID, OUT), lambda i: (0, 0)),
                  pl.BlockSpec((1, OUT), lambda i: (0, 0)),
                  pl.BlockSpec((RB, 1), lambda i: (i, 0))],
        out_specs=pl.BlockSpec((G, OUT), lambda i: (0, 0)),
        out_shape=jax.ShapeDtypeStruct((G, OUT), jnp.float32),
    )(part, inv, h, params['Wl'][li], params['Wr'][li],
      params['bs'][li].reshape(1, HID),
      params['gamma'][li].reshape(1, HID),
      params['beta'][li].reshape(1, HID),
      params['W2'], b2, seg.reshape(N_PAD, 1))
    return pooled
